# baseline (speedup 1.0000x reference)
import functools
import numpy as np
import jax
import jax.numpy as jnp
from jax import lax
from jax.experimental import pallas as pl
from jax.experimental.pallas import tpu as pltpu
from jax.experimental.pallas import tpu_sc as plsc

NUM_CLASSES = 17
KSUB = 6
DIM = 64
PIXEL_SIZE = 150
TEMP = 0.1
BASE_TEMP = 1.0
NA = 2000
NA_PAD = 2048
NPIX = 8 * 128 * 128
NQ = NUM_CLASSES * KSUB * PIXEL_SIZE
NQ_PAD = 15360
NW = 32
APW = NA_PAD // NW
ROW_TILE = 256
N_TILES = NA_PAD // ROW_TILE
QBLK = 1920
N_QBLK = NQ_PAD // QBLK
NEG_BIG = np.float32(-1e30)

_QS = np.array([0.95, 0.85, 0.75, 0.65, 0.55], dtype=np.float32)
_IDXF = (_QS * np.float32(NPIX - 1)).astype(np.float32)
_LOWF = np.floor(_IDXF).astype(np.float32)
_HIW = (_IDXF - _LOWF).astype(np.float32)
_LOW = (np.float32(1.0) - _HIW).astype(np.float32)
_KS = [int(v) for v in _LOWF]


def _sc_gather_body(idx_hbm, aidx_hbm, feats_hbm, pred_hbm, cur_hbm,
                    xraw_hbm, preda_hbm, cura_hbm,
                    idx_v, rows_v, a_v, pred_v, cur_v, sem, sem2):
    c = lax.axis_index("c")
    s = lax.axis_index("s")
    wid = s * 2 + c
    base = wid * APW
    pltpu.sync_copy(aidx_hbm.at[pl.ds(base, APW)], a_v)
    pltpu.async_copy(pred_hbm.at[a_v], pred_v, sem2).wait()
    pltpu.sync_copy(pred_v, preda_hbm.at[pl.ds(base, APW)])
    pltpu.async_copy(cur_hbm.at[a_v], cur_v, sem2).wait()
    pltpu.sync_copy(cur_v, cura_hbm.at[pl.ds(base, APW)])
    pltpu.sync_copy(idx_hbm.at[pl.ds(wid * 32, 32)], idx_v)
    copies = []
    for j in range(32):
        copies.append(pltpu.async_copy(
            feats_hbm.at[idx_v.at[j]], rows_v.at[pl.ds(j * 128, 128)], sem))
    for cp in copies:
        cp.wait()
    pltpu.sync_copy(rows_v, xraw_hbm.at[pl.ds(wid * 4096, 4096)])


_SC_GATHER_CACHE = []


def _get_sc_gather():
    if not _SC_GATHER_CACHE:
        _SC_GATHER_CACHE.append(functools.partial(
            pl.kernel,
            mesh=plsc.VectorSubcoreMesh(
                core_axis_name="c", subcore_axis_name="s"),
            out_type=[
                jax.ShapeDtypeStruct((NA_PAD * DIM,), jnp.float32),
                jax.ShapeDtypeStruct((NA_PAD,), jnp.int32),
                jax.ShapeDtypeStruct((NA_PAD,), jnp.float32),
            ],
            scratch_types=[
                pltpu.VMEM((32, 128), jnp.int32),
                pltpu.VMEM((4096,), jnp.float32),
                pltpu.VMEM((APW,), jnp.int32),
                pltpu.VMEM((APW,), jnp.int32),
                pltpu.VMEM((APW,), jnp.float32),
                pltpu.SemaphoreType.DMA,
                pltpu.SemaphoreType.DMA,
            ],
        )(_sc_gather_body))
    return _SC_GATHER_CACHE[0]


def _prep_body(cur_ref, xraw_ref, preda_ref, cura_ref,
               x_ref, y_ref, pos_ref):
    cur_i = lax.bitcast_convert_type(cur_ref[...], jnp.int32)

    def bs_step(_, carry):
        los, his = carry
        nlos, nhis = [], []
        for t in range(5):
            mid = los[t] + (his[t] - los[t]) // 2
            cnt = jnp.sum((cur_i < mid).astype(jnp.int32))
            ok = cnt <= _KS[t]
            nlos.append(jnp.where(ok, mid, los[t]))
            nhis.append(jnp.where(ok, his[t], mid))
        return tuple(nlos), tuple(nhis)

    zero = jnp.int32(0)
    top = jnp.int32(0x7F800000)
    los, his = lax.fori_loop(
        0, 31, bs_step,
        (tuple(zero for _ in range(5)), tuple(top for _ in range(5))))

    ths = []
    for t in range(5):
        vk = los[t]
        cnt_le = jnp.sum((cur_i <= vk).astype(jnp.int32))
        above = jnp.where(cur_i > vk, cur_i, jnp.int32(0x7F800000))
        vk1 = jnp.where(cnt_le >= _KS[t] + 2, vk, jnp.min(above))
        vkf = lax.bitcast_convert_type(vk, jnp.float32)
        vk1f = lax.bitcast_convert_type(vk1, jnp.float32)
        ths.append(vkf * np.float32(_LOW[t]) + vk1f * np.float32(_HIW[t]))

    c = cura_ref[...]
    sub = jnp.zeros(c.shape, jnp.int32)
    for t in range(5):
        sub = sub + (c < ths[t]).astype(jnp.int32)
    ig = (lax.broadcasted_iota(jnp.int32, c.shape, 0) * 128 +
          lax.broadcasted_iota(jnp.int32, c.shape, 1))
    valid = ig < NA
    y = preda_ref[...] * KSUB + sub
    y = jnp.where(valid, y, -1)
    pos = jnp.where(valid, y * PIXEL_SIZE + ig % PIXEL_SIZE, 0)
    y_ref[...] = y
    pos_ref[...] = pos

    x = xraw_ref[...]
    nrm = jnp.sqrt(jnp.sum(x * x, axis=1, keepdims=True))
    xn = x / (nrm + np.float32(1e-12))
    rvalid = lax.broadcasted_iota(jnp.int32, x.shape, 0) < NA
    x_ref[...] = jnp.where(rvalid, xn, 0.0)


def _run_prep(cur2d, xraw, preda2d, cura2d):
    return pl.pallas_call(
        _prep_body,
        out_shape=[
            jax.ShapeDtypeStruct((NA_PAD, DIM), jnp.float32),
            jax.ShapeDtypeStruct((16, 128), jnp.int32),
            jax.ShapeDtypeStruct((16, 128), jnp.int32),
        ],
    )(cur2d, xraw, preda2d, cura2d)


def _loss_body(pos_ref, x_ref, q_ref, cc_ref, yrow_ref, ycol_ref,
               out_ref, qmem, acc):
    pid = pl.program_id(0)
    inv_t = np.float32(1.0 / TEMP)

    @pl.when(pid == 0)
    def _init():
        qmem[pl.ds(0, NQ), :] = q_ref[...]
        qmem[pl.ds(NQ, NQ_PAD - NQ), :] = jnp.zeros(
            (NQ_PAD - NQ, DIM), jnp.float32)

        def scatter(i, _):
            p = pos_ref[i]
            qmem[pl.ds(p, 1), :] = x_ref[pl.ds(i, 1), :]
            return 0

        lax.fori_loop(0, NA, scatter, 0)
        acc[0] = 0.0
        acc[1] = 0.0
        acc[2] = 0.0

    r = x_ref[pl.ds(pid * ROW_TILE, ROW_TILE), :] * inv_t
    yr = ycol_ref[pl.ds(pid * ROW_TILE, ROW_TILE), :]
    rowid = pid * ROW_TILE + lax.broadcasted_iota(jnp.int32, (ROW_TILE, 1), 0)
    rvalid = (rowid < NA).astype(jnp.float32)

    a = lax.dot_general(r, x_ref[...], (((1,), (1,)), ((), ())),
                        preferred_element_type=jnp.float32)
    colid1 = lax.broadcasted_iota(jnp.int32, (1, NA_PAD), 1)
    colid = lax.broadcasted_iota(jnp.int32, (ROW_TILE, NA_PAD), 1)
    am = jnp.where(colid1 < NA, a, NEG_BIG)
    m1 = jnp.max(am, axis=1, keepdims=True)
    e1 = jnp.exp(am - m1)
    maskf = (yr == yrow_ref[...]).astype(jnp.float32)
    rowid_b = pid * ROW_TILE + lax.broadcasted_iota(
        jnp.int32, (ROW_TILE, NA_PAD), 0)
    eye = (rowid_b == colid).astype(jnp.float32)
    mask_pos = maskf * (1.0 - eye)
    neg_raw = jnp.sum(e1 * (1.0 - maskf), axis=1, keepdims=True)
    denom = jnp.sum(mask_pos, axis=1, keepdims=True)

    NGRP = QBLK // PIXEL_SIZE + 4
    LOG2E = np.float32(1.4426950408889634)
    macc = jnp.full((ROW_TILE, 128), NEG_BIG, jnp.float32)
    total = jnp.zeros((ROW_TILE, 1), jnp.float32)
    matched = jnp.zeros((ROW_TILE, 1), jnp.float32)
    for cb in range(N_QBLK):
        qc = lax.dot_general(
            r, qmem[pl.ds(cb * QBLK, QBLK), :], (((1,), (1,)), ((), ())),
            preferred_element_type=jnp.float32)
        if cb == N_QBLK - 1:
            cgl = cb * QBLK + lax.broadcasted_iota(jnp.int32, (1, QBLK), 1)
            qc = jnp.where(cgl < NQ, qc, NEG_BIG)
        for k in range(QBLK // 128):
            macc = jnp.maximum(macc, qc[:, k * 128:(k + 1) * 128])
        e = jnp.exp2(qc * LOG2E).astype(jnp.bfloat16)
        gbase = (cb * QBLK) // PIXEL_SIZE
        grow = ((cb * QBLK +
                 lax.broadcasted_iota(jnp.int32, (QBLK, NGRP), 0))
                // PIXEL_SIZE - gbase)
        gsel = (grow == lax.broadcasted_iota(
            jnp.int32, (QBLK, NGRP), 1)).astype(jnp.bfloat16)
        s = lax.dot_general(e, gsel, (((1,), (0,)), ((), ())),
                            preferred_element_type=jnp.float32)
        gg = gbase + lax.broadcasted_iota(jnp.int32, (1, NGRP), 1)
        matched = matched + jnp.sum(jnp.where(yr == gg, s, 0.0),
                                    axis=1, keepdims=True)
        total = total + jnp.sum(s, axis=1, keepdims=True)
    m_run = jnp.max(macc, axis=1, keepdims=True)
    neg_logits = (total - matched) * jnp.exp(-m_run)

    lp = (am - m1) - jnp.log(e1 + neg_logits + neg_raw)
    mlpp = jnp.sum(mask_pos * lp, axis=1, keepdims=True) / jnp.maximum(denom, 1.0)
    validr = (denom > 0).astype(jnp.float32)
    ppc_num = jnp.sum(rvalid * validr * mlpp)
    ppc_cnt = jnp.sum(rvalid * validr)

    a2 = lax.dot_general(r, cc_ref[...], (((1,), (1,)), ((), ())),
                         preferred_element_type=jnp.float32,
                         precision=lax.Precision.HIGHEST)
    c2 = lax.broadcasted_iota(jnp.int32, (1, NUM_CLASSES * KSUB), 1)
    m2 = jnp.max(a2, axis=1, keepdims=True)
    l2 = a2 - m2
    e2 = jnp.exp(l2)
    mask2 = (yr == c2).astype(jnp.float32)
    neg2 = jnp.sum((1.0 - mask2) * e2, axis=1, keepdims=True)
    lp2 = l2 - jnp.log(e2 + neg2)
    d2 = jnp.sum(mask2, axis=1, keepdims=True)
    mlpp2 = jnp.sum(mask2 * lp2, axis=1, keepdims=True) / jnp.maximum(d2, 1.0)
    pcc_num = jnp.sum(rvalid * mlpp2)

    acc[0] += ppc_num
    acc[1] += ppc_cnt
    acc[2] += pcc_num

    @pl.when(pid == N_TILES - 1)
    def _fin():
        scale = np.float32(TEMP / BASE_TEMP)
        loss = (-scale * acc[0] / jnp.maximum(acc[1], 1.0)
                - scale * acc[2] / np.float32(NA))
        out_ref[...] = jnp.full((1, 1), loss, jnp.float32)


def _run_loss(pos1d, x, qpad, ccpad, yrow, ycol):
    grid_spec = pltpu.PrefetchScalarGridSpec(
        num_scalar_prefetch=1,
        grid=(N_TILES,),
        in_specs=[
            pl.BlockSpec((NA_PAD, DIM), lambda i, pos: (0, 0)),
            pl.BlockSpec((NQ, DIM), lambda i, pos: (0, 0)),
            pl.BlockSpec((NUM_CLASSES * KSUB, DIM), lambda i, pos: (0, 0)),
            pl.BlockSpec((1, NA_PAD), lambda i, pos: (0, 0)),
            pl.BlockSpec((NA_PAD, 1), lambda i, pos: (0, 0)),
        ],
        out_specs=pl.BlockSpec((1, 1), lambda i, pos: (0, 0)),
        scratch_shapes=[
            pltpu.VMEM((NQ_PAD, DIM), jnp.float32),
            pltpu.SMEM((4,), jnp.float32),
        ],
    )
    return pl.pallas_call(
        _loss_body,
        grid_spec=grid_spec,
        out_shape=jax.ShapeDtypeStruct((1, 1), jnp.float32),
    )(pos1d, x, qpad, ccpad, yrow, ycol)


def kernel(feats, labels, predict, cur, point_queue, cluster_center,
           anchor_idx):
    del labels
    aidx = anchor_idx.astype(jnp.int32)
    aidx = jnp.concatenate(
        [aidx, jnp.zeros((NA_PAD - NA,), jnp.int32)])
    base = (aidx // 16384) * (DIM * 16384) + (aidx % 16384)
    fidx = (base[:, None] + jnp.arange(DIM, dtype=jnp.int32)[None, :] * 16384)
    fidx = fidx.reshape(NA_PAD * DIM // 128, 128)

    feats_flat = feats.reshape(-1)
    pred_flat = predict.reshape(-1).astype(jnp.int32)
    cur_flat = cur.reshape(-1)

    xraw_flat, preda, cura = _get_sc_gather()(
        fidx, aidx, feats_flat, pred_flat, cur_flat)

    x, y2d, pos2d = _run_prep(
        cur_flat.reshape(1024, 128),
        xraw_flat.reshape(NA_PAD, DIM),
        preda.reshape(16, 128),
        cura.reshape(16, 128))

    loss = _run_loss(
        pos2d.reshape(NA_PAD),
        x,
        point_queue.reshape(NQ, DIM),
        cluster_center.reshape(NUM_CLASSES * KSUB, DIM),
        y2d.reshape(1, NA_PAD),
        y2d.reshape(NA_PAD, 1))
    return loss[0, 0]


# speedup vs baseline: 1.0332x; 1.0332x over previous
import functools
import numpy as np
import jax
import jax.numpy as jnp
from jax import lax
from jax.experimental import pallas as pl
from jax.experimental.pallas import tpu as pltpu
from jax.experimental.pallas import tpu_sc as plsc

NUM_CLASSES = 17
KSUB = 6
DIM = 64
PIXEL_SIZE = 150
TEMP = 0.1
BASE_TEMP = 1.0
NA = 2000
NA_PAD = 2048
NPIX = 8 * 128 * 128
NQ = NUM_CLASSES * KSUB * PIXEL_SIZE
NQ_PAD = 15360
NW = 32
APW = NA_PAD // NW
ROW_TILE = 256
N_TILES = NA_PAD // ROW_TILE
QBLK = 1920
N_QBLK = NQ_PAD // QBLK
NEG_BIG = np.float32(-1e30)

_QS = np.array([0.95, 0.85, 0.75, 0.65, 0.55], dtype=np.float32)
_IDXF = (_QS * np.float32(NPIX - 1)).astype(np.float32)
_LOWF = np.floor(_IDXF).astype(np.float32)
_HIW = (_IDXF - _LOWF).astype(np.float32)
_LOW = (np.float32(1.0) - _HIW).astype(np.float32)
_KS = [int(v) for v in _LOWF]


def _sc_gather_body(idx_hbm, aidx_hbm, feats_hbm, pred_hbm, cur_hbm,
                    xraw_hbm, preda_hbm, cura_hbm,
                    idx_v, rows_v, a_v, pred_v, cur_v, sem, sem2):
    c = lax.axis_index("c")
    s = lax.axis_index("s")
    wid = s * 2 + c
    base = wid * APW
    pltpu.sync_copy(aidx_hbm.at[pl.ds(base, APW)], a_v)
    pltpu.async_copy(pred_hbm.at[a_v], pred_v, sem2).wait()
    pltpu.sync_copy(pred_v, preda_hbm.at[pl.ds(base, APW)])
    pltpu.async_copy(cur_hbm.at[a_v], cur_v, sem2).wait()
    pltpu.sync_copy(cur_v, cura_hbm.at[pl.ds(base, APW)])
    pltpu.sync_copy(idx_hbm.at[pl.ds(wid * 32, 32)], idx_v)
    copies = []
    for j in range(32):
        copies.append(pltpu.async_copy(
            feats_hbm.at[idx_v.at[j]], rows_v.at[pl.ds(j * 128, 128)], sem))
    for cp in copies:
        cp.wait()
    pltpu.sync_copy(rows_v, xraw_hbm.at[pl.ds(wid * 4096, 4096)])


_SC_GATHER_CACHE = []


def _get_sc_gather():
    if not _SC_GATHER_CACHE:
        _SC_GATHER_CACHE.append(functools.partial(
            pl.kernel,
            mesh=plsc.VectorSubcoreMesh(
                core_axis_name="c", subcore_axis_name="s"),
            out_type=[
                jax.ShapeDtypeStruct((NA_PAD * DIM,), jnp.float32),
                jax.ShapeDtypeStruct((NA_PAD,), jnp.int32),
                jax.ShapeDtypeStruct((NA_PAD,), jnp.float32),
            ],
            scratch_types=[
                pltpu.VMEM((32, 128), jnp.int32),
                pltpu.VMEM((4096,), jnp.float32),
                pltpu.VMEM((APW,), jnp.int32),
                pltpu.VMEM((APW,), jnp.int32),
                pltpu.VMEM((APW,), jnp.float32),
                pltpu.SemaphoreType.DMA,
                pltpu.SemaphoreType.DMA,
            ],
        )(_sc_gather_body))
    return _SC_GATHER_CACHE[0]


def _prep_body(cur_ref, xraw_ref, preda_ref, cura_ref,
               x_ref, y_ref, pos_ref):
    cur_i = lax.bitcast_convert_type(cur_ref[...], jnp.int32)

    def bs_step(_, carry):
        los, his = carry
        nlos, nhis = [], []
        for t in range(5):
            mid = los[t] + (his[t] - los[t]) // 2
            cnt = jnp.sum((cur_i < mid).astype(jnp.int32))
            ok = cnt <= _KS[t]
            nlos.append(jnp.where(ok, mid, los[t]))
            nhis.append(jnp.where(ok, his[t], mid))
        return tuple(nlos), tuple(nhis)

    zero = jnp.int32(0)
    top = jnp.int32(0x7F800000)
    los, his = lax.fori_loop(
        0, 31, bs_step,
        (tuple(zero for _ in range(5)), tuple(top for _ in range(5))))

    ths = []
    for t in range(5):
        vk = los[t]
        cnt_le = jnp.sum((cur_i <= vk).astype(jnp.int32))
        above = jnp.where(cur_i > vk, cur_i, jnp.int32(0x7F800000))
        vk1 = jnp.where(cnt_le >= _KS[t] + 2, vk, jnp.min(above))
        vkf = lax.bitcast_convert_type(vk, jnp.float32)
        vk1f = lax.bitcast_convert_type(vk1, jnp.float32)
        ths.append(vkf * np.float32(_LOW[t]) + vk1f * np.float32(_HIW[t]))

    c = cura_ref[...]
    sub = jnp.zeros(c.shape, jnp.int32)
    for t in range(5):
        sub = sub + (c < ths[t]).astype(jnp.int32)
    ig = (lax.broadcasted_iota(jnp.int32, c.shape, 0) * 128 +
          lax.broadcasted_iota(jnp.int32, c.shape, 1))
    valid = ig < NA
    y = preda_ref[...] * KSUB + sub
    y = jnp.where(valid, y, -1)
    pos = jnp.where(valid, y * PIXEL_SIZE + ig % PIXEL_SIZE, 0)
    y_ref[...] = y
    pos_ref[...] = pos

    x = xraw_ref[...]
    nrm = jnp.sqrt(jnp.sum(x * x, axis=1, keepdims=True))
    xn = x / (nrm + np.float32(1e-12))
    rvalid = lax.broadcasted_iota(jnp.int32, x.shape, 0) < NA
    x_ref[...] = jnp.where(rvalid, xn, 0.0)


def _run_prep(cur2d, xraw, preda2d, cura2d):
    return pl.pallas_call(
        _prep_body,
        out_shape=[
            jax.ShapeDtypeStruct((NA_PAD, DIM), jnp.float32),
            jax.ShapeDtypeStruct((16, 128), jnp.int32),
            jax.ShapeDtypeStruct((16, 128), jnp.int32),
        ],
    )(cur2d, xraw, preda2d, cura2d)


def _loss_body(pos_ref, x_ref, q_ref, cc_ref, yrow_ref, ycol_ref,
               out_ref, qmem, acc):
    pid = pl.program_id(0)
    inv_t = np.float32(1.0 / TEMP)

    @pl.when(pid == 0)
    def _init():
        qmem[pl.ds(0, NQ), :] = q_ref[...]
        qmem[pl.ds(NQ, NQ_PAD - NQ), :] = jnp.zeros(
            (NQ_PAD - NQ, DIM), jnp.float32)

        def scatter(i, _):
            p = pos_ref[i]
            qmem[pl.ds(p, 1), :] = x_ref[pl.ds(i, 1), :]
            return 0

        lax.fori_loop(0, NA, scatter, 0)
        acc[0] = 0.0
        acc[1] = 0.0
        acc[2] = 0.0

    r = x_ref[pl.ds(pid * ROW_TILE, ROW_TILE), :] * inv_t
    yr = ycol_ref[pl.ds(pid * ROW_TILE, ROW_TILE), :]
    rowid = pid * ROW_TILE + lax.broadcasted_iota(jnp.int32, (ROW_TILE, 1), 0)
    rvalid = (rowid < NA).astype(jnp.float32)

    a = lax.dot_general(r, x_ref[...], (((1,), (1,)), ((), ())),
                        preferred_element_type=jnp.float32)
    colid1 = lax.broadcasted_iota(jnp.int32, (1, NA_PAD), 1)
    colid = lax.broadcasted_iota(jnp.int32, (ROW_TILE, NA_PAD), 1)
    am = jnp.where(colid1 < NA, a, NEG_BIG)
    m1 = jnp.max(am, axis=1, keepdims=True)
    e1 = jnp.exp(am - m1)
    maskf = (yr == yrow_ref[...]).astype(jnp.float32)
    rowid_b = pid * ROW_TILE + lax.broadcasted_iota(
        jnp.int32, (ROW_TILE, NA_PAD), 0)
    eye = (rowid_b == colid).astype(jnp.float32)
    mask_pos = maskf * (1.0 - eye)
    neg_raw = jnp.sum(e1 * (1.0 - maskf), axis=1, keepdims=True)
    denom = jnp.sum(mask_pos, axis=1, keepdims=True)

    NGRP = QBLK // PIXEL_SIZE + 4
    LOG2E = np.float32(1.4426950408889634)
    m_run = jnp.full((ROW_TILE, 1), NEG_BIG, jnp.float32)
    total = jnp.zeros((ROW_TILE, 1), jnp.float32)
    matched = jnp.zeros((ROW_TILE, 1), jnp.float32)
    for cb in range(N_QBLK):
        qc = lax.dot_general(
            r, qmem[pl.ds(cb * QBLK, QBLK), :], (((1,), (1,)), ((), ())),
            preferred_element_type=jnp.float32)
        if cb == N_QBLK - 1:
            cgl = cb * QBLK + lax.broadcasted_iota(jnp.int32, (1, QBLK), 1)
            qc = jnp.where(cgl < NQ, qc, NEG_BIG)
        m_run = jnp.maximum(m_run, jnp.max(qc, axis=1, keepdims=True))
        e = jnp.exp2(qc * LOG2E)
        gbase = (cb * QBLK) // PIXEL_SIZE
        grow = ((cb * QBLK +
                 lax.broadcasted_iota(jnp.int32, (QBLK, NGRP), 0))
                // PIXEL_SIZE - gbase)
        gsel = (grow == lax.broadcasted_iota(
            jnp.int32, (QBLK, NGRP), 1)).astype(jnp.float32)
        s = lax.dot_general(e, gsel, (((1,), (0,)), ((), ())),
                            preferred_element_type=jnp.float32)
        gg = gbase + lax.broadcasted_iota(jnp.int32, (1, NGRP), 1)
        matched = matched + jnp.sum(jnp.where(yr == gg, s, 0.0),
                                    axis=1, keepdims=True)
        total = total + jnp.sum(s, axis=1, keepdims=True)
    neg_logits = (total - matched) * jnp.exp(-m_run)

    lp = (am - m1) - jnp.log(e1 + neg_logits + neg_raw)
    mlpp = jnp.sum(mask_pos * lp, axis=1, keepdims=True) / jnp.maximum(denom, 1.0)
    validr = (denom > 0).astype(jnp.float32)
    ppc_num = jnp.sum(rvalid * validr * mlpp)
    ppc_cnt = jnp.sum(rvalid * validr)

    a2 = lax.dot_general(r, cc_ref[...], (((1,), (1,)), ((), ())),
                         preferred_element_type=jnp.float32,
                         precision=lax.Precision.HIGHEST)
    c2 = lax.broadcasted_iota(jnp.int32, (1, NUM_CLASSES * KSUB), 1)
    m2 = jnp.max(a2, axis=1, keepdims=True)
    l2 = a2 - m2
    e2 = jnp.exp(l2)
    mask2 = (yr == c2).astype(jnp.float32)
    neg2 = jnp.sum((1.0 - mask2) * e2, axis=1, keepdims=True)
    lp2 = l2 - jnp.log(e2 + neg2)
    d2 = jnp.sum(mask2, axis=1, keepdims=True)
    mlpp2 = jnp.sum(mask2 * lp2, axis=1, keepdims=True) / jnp.maximum(d2, 1.0)
    pcc_num = jnp.sum(rvalid * mlpp2)

    acc[0] += ppc_num
    acc[1] += ppc_cnt
    acc[2] += pcc_num

    @pl.when(pid == N_TILES - 1)
    def _fin():
        scale = np.float32(TEMP / BASE_TEMP)
        loss = (-scale * acc[0] / jnp.maximum(acc[1], 1.0)
                - scale * acc[2] / np.float32(NA))
        out_ref[...] = jnp.full((1, 1), loss, jnp.float32)


def _run_loss(pos1d, x, qpad, ccpad, yrow, ycol):
    grid_spec = pltpu.PrefetchScalarGridSpec(
        num_scalar_prefetch=1,
        grid=(N_TILES,),
        in_specs=[
            pl.BlockSpec((NA_PAD, DIM), lambda i, pos: (0, 0)),
            pl.BlockSpec((NQ, DIM), lambda i, pos: (0, 0)),
            pl.BlockSpec((NUM_CLASSES * KSUB, DIM), lambda i, pos: (0, 0)),
            pl.BlockSpec((1, NA_PAD), lambda i, pos: (0, 0)),
            pl.BlockSpec((NA_PAD, 1), lambda i, pos: (0, 0)),
        ],
        out_specs=pl.BlockSpec((1, 1), lambda i, pos: (0, 0)),
        scratch_shapes=[
            pltpu.VMEM((NQ_PAD, DIM), jnp.float32),
            pltpu.SMEM((4,), jnp.float32),
        ],
    )
    return pl.pallas_call(
        _loss_body,
        grid_spec=grid_spec,
        out_shape=jax.ShapeDtypeStruct((1, 1), jnp.float32),
    )(pos1d, x, qpad, ccpad, yrow, ycol)


def kernel(feats, labels, predict, cur, point_queue, cluster_center,
           anchor_idx):
    del labels
    aidx = anchor_idx.astype(jnp.int32)
    aidx = jnp.concatenate(
        [aidx, jnp.zeros((NA_PAD - NA,), jnp.int32)])
    base = (aidx // 16384) * (DIM * 16384) + (aidx % 16384)
    fidx = (base[:, None] + jnp.arange(DIM, dtype=jnp.int32)[None, :] * 16384)
    fidx = fidx.reshape(NA_PAD * DIM // 128, 128)

    feats_flat = feats.reshape(-1)
    pred_flat = predict.reshape(-1).astype(jnp.int32)
    cur_flat = cur.reshape(-1)

    xraw_flat, preda, cura = _get_sc_gather()(
        fidx, aidx, feats_flat, pred_flat, cur_flat)

    x, y2d, pos2d = _run_prep(
        cur_flat.reshape(1024, 128),
        xraw_flat.reshape(NA_PAD, DIM),
        preda.reshape(16, 128),
        cura.reshape(16, 128))

    loss = _run_loss(
        pos2d.reshape(NA_PAD),
        x,
        point_queue.reshape(NQ, DIM),
        cluster_center.reshape(NUM_CLASSES * KSUB, DIM),
        y2d.reshape(1, NA_PAD),
        y2d.reshape(NA_PAD, 1))
    return loss[0, 0]


# speedup vs baseline: 1.0360x; 1.0027x over previous
import functools
import numpy as np
import jax
import jax.numpy as jnp
from jax import lax
from jax.experimental import pallas as pl
from jax.experimental.pallas import tpu as pltpu
from jax.experimental.pallas import tpu_sc as plsc

NUM_CLASSES = 17
KSUB = 6
DIM = 64
PIXEL_SIZE = 150
TEMP = 0.1
BASE_TEMP = 1.0
NA = 2000
NA_PAD = 2048
NPIX = 8 * 128 * 128
NQ = NUM_CLASSES * KSUB * PIXEL_SIZE
NQ_PAD = 15360
NW = 32
APW = NA_PAD // NW
ROW_TILE = 256
N_TILES = NA_PAD // ROW_TILE
QBLK = 1920
N_QBLK = NQ_PAD // QBLK
NEG_BIG = np.float32(-1e30)

_QS = np.array([0.95, 0.85, 0.75, 0.65, 0.55], dtype=np.float32)
_IDXF = (_QS * np.float32(NPIX - 1)).astype(np.float32)
_LOWF = np.floor(_IDXF).astype(np.float32)
_HIW = (_IDXF - _LOWF).astype(np.float32)
_LOW = (np.float32(1.0) - _HIW).astype(np.float32)
_KS = [int(v) for v in _LOWF]


def _sc_gather_body(idx_hbm, aidx_hbm, feats_hbm, pred_hbm, cur_hbm,
                    xraw_hbm, preda_hbm, cura_hbm,
                    idx_v, rows_v, a_v, pred_v, cur_v, sem, sem2):
    c = lax.axis_index("c")
    s = lax.axis_index("s")
    wid = s * 2 + c
    base = wid * APW
    pltpu.sync_copy(aidx_hbm.at[pl.ds(base, APW)], a_v)
    pltpu.async_copy(pred_hbm.at[a_v], pred_v, sem2).wait()
    pltpu.sync_copy(pred_v, preda_hbm.at[pl.ds(base, APW)])
    pltpu.async_copy(cur_hbm.at[a_v], cur_v, sem2).wait()
    pltpu.sync_copy(cur_v, cura_hbm.at[pl.ds(base, APW)])
    pltpu.sync_copy(idx_hbm.at[pl.ds(wid * 32, 32)], idx_v)
    copies = []
    for j in range(32):
        copies.append(pltpu.async_copy(
            feats_hbm.at[idx_v.at[j]], rows_v.at[pl.ds(j * 128, 128)], sem))
    for cp in copies:
        cp.wait()
    pltpu.sync_copy(rows_v, xraw_hbm.at[pl.ds(wid * 4096, 4096)])


_SC_GATHER_CACHE = []


def _get_sc_gather():
    if not _SC_GATHER_CACHE:
        _SC_GATHER_CACHE.append(functools.partial(
            pl.kernel,
            mesh=plsc.VectorSubcoreMesh(
                core_axis_name="c", subcore_axis_name="s"),
            out_type=[
                jax.ShapeDtypeStruct((NA_PAD * DIM,), jnp.float32),
                jax.ShapeDtypeStruct((NA_PAD,), jnp.int32),
                jax.ShapeDtypeStruct((NA_PAD,), jnp.float32),
            ],
            scratch_types=[
                pltpu.VMEM((32, 128), jnp.int32),
                pltpu.VMEM((4096,), jnp.float32),
                pltpu.VMEM((APW,), jnp.int32),
                pltpu.VMEM((APW,), jnp.int32),
                pltpu.VMEM((APW,), jnp.float32),
                pltpu.SemaphoreType.DMA,
                pltpu.SemaphoreType.DMA,
            ],
        )(_sc_gather_body))
    return _SC_GATHER_CACHE[0]


def _quant_body(cur_ref, ths_ref):
    cur_i = lax.bitcast_convert_type(cur_ref[...], jnp.int32)

    def bs_step(_, carry):
        los, his = carry
        nlos, nhis = [], []
        for t in range(5):
            mid = los[t] + (his[t] - los[t]) // 2
            cnt = jnp.sum((cur_i < mid).astype(jnp.int32))
            ok = cnt <= _KS[t]
            nlos.append(jnp.where(ok, mid, los[t]))
            nhis.append(jnp.where(ok, his[t], mid))
        return tuple(nlos), tuple(nhis)

    zero = jnp.int32(0)
    top = jnp.int32(0x7F800000)
    los, his = lax.fori_loop(
        0, 31, bs_step,
        (tuple(zero for _ in range(5)), tuple(top for _ in range(5))))

    for t in range(5):
        vk = los[t]
        cnt_le = jnp.sum((cur_i <= vk).astype(jnp.int32))
        above = jnp.where(cur_i > vk, cur_i, jnp.int32(0x7F800000))
        vk1 = jnp.where(cnt_le >= _KS[t] + 2, vk, jnp.min(above))
        vkf = lax.bitcast_convert_type(vk, jnp.float32)
        vk1f = lax.bitcast_convert_type(vk1, jnp.float32)
        ths_ref[t] = vkf * np.float32(_LOW[t]) + vk1f * np.float32(_HIW[t])


def _run_quant(cur2d):
    return pl.pallas_call(
        _quant_body,
        out_shape=jax.ShapeDtypeStruct((8,), jnp.float32),
        out_specs=pl.BlockSpec(memory_space=pltpu.MemorySpace.SMEM),
    )(cur2d)


def _prep_body(ths_ref, xraw_ref, preda_ref, cura_ref,
               x_ref, y_ref, pos_ref):
    c = cura_ref[...]
    sub = jnp.zeros(c.shape, jnp.int32)
    for t in range(5):
        sub = sub + (c < ths_ref[t]).astype(jnp.int32)
    ig = (lax.broadcasted_iota(jnp.int32, c.shape, 0) * 128 +
          lax.broadcasted_iota(jnp.int32, c.shape, 1))
    valid = ig < NA
    y = preda_ref[...] * KSUB + sub
    y = jnp.where(valid, y, -1)
    pos = jnp.where(valid, y * PIXEL_SIZE + ig % PIXEL_SIZE, 0)
    y_ref[...] = y
    pos_ref[...] = pos

    x = xraw_ref[...]
    nrm = jnp.sqrt(jnp.sum(x * x, axis=1, keepdims=True))
    xn = x / (nrm + np.float32(1e-12))
    rvalid = lax.broadcasted_iota(jnp.int32, x.shape, 0) < NA
    x_ref[...] = jnp.where(rvalid, xn, 0.0)


def _run_prep(ths, xraw, preda2d, cura2d):
    return pl.pallas_call(
        _prep_body,
        out_shape=[
            jax.ShapeDtypeStruct((NA_PAD, DIM), jnp.float32),
            jax.ShapeDtypeStruct((16, 128), jnp.int32),
            jax.ShapeDtypeStruct((16, 128), jnp.int32),
        ],
        in_specs=[
            pl.BlockSpec(memory_space=pltpu.MemorySpace.SMEM),
            pl.BlockSpec(memory_space=pltpu.MemorySpace.VMEM),
            pl.BlockSpec(memory_space=pltpu.MemorySpace.VMEM),
            pl.BlockSpec(memory_space=pltpu.MemorySpace.VMEM),
        ],
    )(ths, xraw, preda2d, cura2d)


def _loss_body(pos_ref, x_ref, q_ref, cc_ref, yrow_ref, ycol_ref,
               out_ref, qmem, acc):
    pid = pl.program_id(0)
    inv_t = np.float32(1.0 / TEMP)

    @pl.when(pid == 0)
    def _init():
        qmem[pl.ds(0, NQ), :] = q_ref[...]
        qmem[pl.ds(NQ, NQ_PAD - NQ), :] = jnp.zeros(
            (NQ_PAD - NQ, DIM), jnp.float32)

        def scatter(i, _):
            p = pos_ref[i]
            qmem[pl.ds(p, 1), :] = x_ref[pl.ds(i, 1), :]
            return 0

        lax.fori_loop(0, NA, scatter, 0)
        acc[0] = 0.0
        acc[1] = 0.0
        acc[2] = 0.0

    r = x_ref[pl.ds(pid * ROW_TILE, ROW_TILE), :] * inv_t
    yr = ycol_ref[pl.ds(pid * ROW_TILE, ROW_TILE), :]
    rowid = pid * ROW_TILE + lax.broadcasted_iota(jnp.int32, (ROW_TILE, 1), 0)
    rvalid = (rowid < NA).astype(jnp.float32)

    a = lax.dot_general(r, x_ref[...], (((1,), (1,)), ((), ())),
                        preferred_element_type=jnp.float32)
    colid1 = lax.broadcasted_iota(jnp.int32, (1, NA_PAD), 1)
    colid = lax.broadcasted_iota(jnp.int32, (ROW_TILE, NA_PAD), 1)
    am = jnp.where(colid1 < NA, a, NEG_BIG)
    m1 = jnp.max(am, axis=1, keepdims=True)
    e1 = jnp.exp(am - m1)
    maskf = (yr == yrow_ref[...]).astype(jnp.float32)
    rowid_b = pid * ROW_TILE + lax.broadcasted_iota(
        jnp.int32, (ROW_TILE, NA_PAD), 0)
    eye = (rowid_b == colid).astype(jnp.float32)
    mask_pos = maskf * (1.0 - eye)
    neg_raw = jnp.sum(e1 * (1.0 - maskf), axis=1, keepdims=True)
    denom = jnp.sum(mask_pos, axis=1, keepdims=True)

    NGRP = QBLK // PIXEL_SIZE + 4
    LOG2E = np.float32(1.4426950408889634)
    m_run = jnp.full((ROW_TILE, 1), NEG_BIG, jnp.float32)
    total = jnp.zeros((ROW_TILE, 1), jnp.float32)
    matched = jnp.zeros((ROW_TILE, 1), jnp.float32)
    for cb in range(N_QBLK):
        qc = lax.dot_general(
            r, qmem[pl.ds(cb * QBLK, QBLK), :], (((1,), (1,)), ((), ())),
            preferred_element_type=jnp.float32)
        if cb == N_QBLK - 1:
            cgl = cb * QBLK + lax.broadcasted_iota(jnp.int32, (1, QBLK), 1)
            qc = jnp.where(cgl < NQ, qc, NEG_BIG)
        m_run = jnp.maximum(m_run, jnp.max(qc, axis=1, keepdims=True))
        e = jnp.exp2(qc * LOG2E)
        gbase = (cb * QBLK) // PIXEL_SIZE
        grow = ((cb * QBLK +
                 lax.broadcasted_iota(jnp.int32, (QBLK, NGRP), 0))
                // PIXEL_SIZE - gbase)
        gsel = (grow == lax.broadcasted_iota(
            jnp.int32, (QBLK, NGRP), 1)).astype(jnp.float32)
        s = lax.dot_general(e, gsel, (((1,), (0,)), ((), ())),
                            preferred_element_type=jnp.float32)
        gg = gbase + lax.broadcasted_iota(jnp.int32, (1, NGRP), 1)
        matched = matched + jnp.sum(jnp.where(yr == gg, s, 0.0),
                                    axis=1, keepdims=True)
        total = total + jnp.sum(s, axis=1, keepdims=True)
    neg_logits = (total - matched) * jnp.exp(-m_run)

    lp = (am - m1) - jnp.log(e1 + neg_logits + neg_raw)
    mlpp = jnp.sum(mask_pos * lp, axis=1, keepdims=True) / jnp.maximum(denom, 1.0)
    validr = (denom > 0).astype(jnp.float32)
    ppc_num = jnp.sum(rvalid * validr * mlpp)
    ppc_cnt = jnp.sum(rvalid * validr)

    a2 = lax.dot_general(r, cc_ref[...], (((1,), (1,)), ((), ())),
                         preferred_element_type=jnp.float32,
                         precision=lax.Precision.HIGHEST)
    c2 = lax.broadcasted_iota(jnp.int32, (1, NUM_CLASSES * KSUB), 1)
    m2 = jnp.max(a2, axis=1, keepdims=True)
    l2 = a2 - m2
    e2 = jnp.exp(l2)
    mask2 = (yr == c2).astype(jnp.float32)
    neg2 = jnp.sum((1.0 - mask2) * e2, axis=1, keepdims=True)
    lp2 = l2 - jnp.log(e2 + neg2)
    d2 = jnp.sum(mask2, axis=1, keepdims=True)
    mlpp2 = jnp.sum(mask2 * lp2, axis=1, keepdims=True) / jnp.maximum(d2, 1.0)
    pcc_num = jnp.sum(rvalid * mlpp2)

    acc[0] += ppc_num
    acc[1] += ppc_cnt
    acc[2] += pcc_num

    @pl.when(pid == N_TILES - 1)
    def _fin():
        scale = np.float32(TEMP / BASE_TEMP)
        loss = (-scale * acc[0] / jnp.maximum(acc[1], 1.0)
                - scale * acc[2] / np.float32(NA))
        out_ref[...] = jnp.full((1, 1), loss, jnp.float32)


def _run_loss(pos1d, x, qpad, ccpad, yrow, ycol):
    grid_spec = pltpu.PrefetchScalarGridSpec(
        num_scalar_prefetch=1,
        grid=(N_TILES,),
        in_specs=[
            pl.BlockSpec((NA_PAD, DIM), lambda i, pos: (0, 0)),
            pl.BlockSpec((NQ, DIM), lambda i, pos: (0, 0)),
            pl.BlockSpec((NUM_CLASSES * KSUB, DIM), lambda i, pos: (0, 0)),
            pl.BlockSpec((1, NA_PAD), lambda i, pos: (0, 0)),
            pl.BlockSpec((NA_PAD, 1), lambda i, pos: (0, 0)),
        ],
        out_specs=pl.BlockSpec((1, 1), lambda i, pos: (0, 0)),
        scratch_shapes=[
            pltpu.VMEM((NQ_PAD, DIM), jnp.float32),
            pltpu.SMEM((4,), jnp.float32),
        ],
    )
    return pl.pallas_call(
        _loss_body,
        grid_spec=grid_spec,
        out_shape=jax.ShapeDtypeStruct((1, 1), jnp.float32),
    )(pos1d, x, qpad, ccpad, yrow, ycol)


def kernel(feats, labels, predict, cur, point_queue, cluster_center,
           anchor_idx):
    del labels
    aidx = anchor_idx.astype(jnp.int32)
    aidx = jnp.concatenate(
        [aidx, jnp.zeros((NA_PAD - NA,), jnp.int32)])
    base = (aidx // 16384) * (DIM * 16384) + (aidx % 16384)
    fidx = (base[:, None] + jnp.arange(DIM, dtype=jnp.int32)[None, :] * 16384)
    fidx = fidx.reshape(NA_PAD * DIM // 128, 128)

    feats_flat = feats.reshape(-1)
    pred_flat = predict.reshape(-1).astype(jnp.int32)
    cur_flat = cur.reshape(-1)

    xraw_flat, preda, cura = _get_sc_gather()(
        fidx, aidx, feats_flat, pred_flat, cur_flat)

    ths = _run_quant(cur_flat.reshape(1024, 128))

    x, y2d, pos2d = _run_prep(
        ths,
        xraw_flat.reshape(NA_PAD, DIM),
        preda.reshape(16, 128),
        cura.reshape(16, 128))

    loss = _run_loss(
        pos2d.reshape(NA_PAD),
        x,
        point_queue.reshape(NQ, DIM),
        cluster_center.reshape(NUM_CLASSES * KSUB, DIM),
        y2d.reshape(1, NA_PAD),
        y2d.reshape(NA_PAD, 1))
    return loss[0, 0]


# speedup vs baseline: 1.0655x; 1.0285x over previous
import functools
import numpy as np
import jax
import jax.numpy as jnp
from jax import lax
from jax.experimental import pallas as pl
from jax.experimental.pallas import tpu as pltpu
from jax.experimental.pallas import tpu_sc as plsc

NUM_CLASSES = 17
KSUB = 6
DIM = 64
PIXEL_SIZE = 150
TEMP = 0.1
BASE_TEMP = 1.0
NA = 2000
NA_PAD = 2048
NPIX = 8 * 128 * 128
NQ = NUM_CLASSES * KSUB * PIXEL_SIZE
NQ_PAD = 15360
NW = 32
APW = NA_PAD // NW
ROW_TILE = 512
N_TILES = NA_PAD // ROW_TILE
QBLK = 1920
N_QBLK = NQ_PAD // QBLK
NEG_BIG = np.float32(-1e30)

_QS = np.array([0.95, 0.85, 0.75, 0.65, 0.55], dtype=np.float32)
_IDXF = (_QS * np.float32(NPIX - 1)).astype(np.float32)
_LOWF = np.floor(_IDXF).astype(np.float32)
_HIW = (_IDXF - _LOWF).astype(np.float32)
_LOW = (np.float32(1.0) - _HIW).astype(np.float32)
_KS = [int(v) for v in _LOWF]


def _sc_gather_body(idx_hbm, aidx_hbm, feats_hbm, pred_hbm, cur_hbm,
                    xraw_hbm, preda_hbm, cura_hbm,
                    idx_v, rows_v, a_v, pred_v, cur_v, sem, sem2):
    c = lax.axis_index("c")
    s = lax.axis_index("s")
    wid = s * 2 + c
    base = wid * APW
    pltpu.sync_copy(aidx_hbm.at[pl.ds(base, APW)], a_v)
    pltpu.async_copy(pred_hbm.at[a_v], pred_v, sem2).wait()
    pltpu.sync_copy(pred_v, preda_hbm.at[pl.ds(base, APW)])
    pltpu.async_copy(cur_hbm.at[a_v], cur_v, sem2).wait()
    pltpu.sync_copy(cur_v, cura_hbm.at[pl.ds(base, APW)])
    pltpu.sync_copy(idx_hbm.at[pl.ds(wid * 32, 32)], idx_v)
    copies = []
    for j in range(32):
        copies.append(pltpu.async_copy(
            feats_hbm.at[idx_v.at[j]], rows_v.at[pl.ds(j * 128, 128)], sem))
    for cp in copies:
        cp.wait()
    pltpu.sync_copy(rows_v, xraw_hbm.at[pl.ds(wid * 4096, 4096)])


_SC_GATHER_CACHE = []


def _get_sc_gather():
    if not _SC_GATHER_CACHE:
        _SC_GATHER_CACHE.append(functools.partial(
            pl.kernel,
            mesh=plsc.VectorSubcoreMesh(
                core_axis_name="c", subcore_axis_name="s"),
            out_type=[
                jax.ShapeDtypeStruct((NA_PAD * DIM,), jnp.float32),
                jax.ShapeDtypeStruct((NA_PAD,), jnp.int32),
                jax.ShapeDtypeStruct((NA_PAD,), jnp.float32),
            ],
            scratch_types=[
                pltpu.VMEM((32, 128), jnp.int32),
                pltpu.VMEM((4096,), jnp.float32),
                pltpu.VMEM((APW,), jnp.int32),
                pltpu.VMEM((APW,), jnp.int32),
                pltpu.VMEM((APW,), jnp.float32),
                pltpu.SemaphoreType.DMA,
                pltpu.SemaphoreType.DMA,
            ],
        )(_sc_gather_body))
    return _SC_GATHER_CACHE[0]


def _quant_body(cur_ref, ths_ref):
    cur_i = lax.bitcast_convert_type(cur_ref[...], jnp.int32)

    def bs_step(_, carry):
        los, his = carry
        nlos, nhis = [], []
        for t in range(5):
            mid = los[t] + (his[t] - los[t]) // 2
            cnt = jnp.sum((cur_i < mid).astype(jnp.int32))
            ok = cnt <= _KS[t]
            nlos.append(jnp.where(ok, mid, los[t]))
            nhis.append(jnp.where(ok, his[t], mid))
        return tuple(nlos), tuple(nhis)

    zero = jnp.int32(0)
    top = jnp.int32(0x7F800000)
    los, his = lax.fori_loop(
        0, 31, bs_step,
        (tuple(zero for _ in range(5)), tuple(top for _ in range(5))))

    for t in range(5):
        vk = los[t]
        cnt_le = jnp.sum((cur_i <= vk).astype(jnp.int32))
        above = jnp.where(cur_i > vk, cur_i, jnp.int32(0x7F800000))
        vk1 = jnp.where(cnt_le >= _KS[t] + 2, vk, jnp.min(above))
        vkf = lax.bitcast_convert_type(vk, jnp.float32)
        vk1f = lax.bitcast_convert_type(vk1, jnp.float32)
        ths_ref[t] = vkf * np.float32(_LOW[t]) + vk1f * np.float32(_HIW[t])


def _run_quant(cur2d):
    return pl.pallas_call(
        _quant_body,
        out_shape=jax.ShapeDtypeStruct((8,), jnp.float32),
        out_specs=pl.BlockSpec(memory_space=pltpu.MemorySpace.SMEM),
    )(cur2d)


def _prep_body(ths_ref, xraw_ref, preda_ref, cura_ref,
               x_ref, y_ref, pos_ref):
    c = cura_ref[...]
    sub = jnp.zeros(c.shape, jnp.int32)
    for t in range(5):
        sub = sub + (c < ths_ref[t]).astype(jnp.int32)
    ig = (lax.broadcasted_iota(jnp.int32, c.shape, 0) * 128 +
          lax.broadcasted_iota(jnp.int32, c.shape, 1))
    valid = ig < NA
    y = preda_ref[...] * KSUB + sub
    y = jnp.where(valid, y, -1)
    pos = jnp.where(valid, y * PIXEL_SIZE + ig % PIXEL_SIZE, 0)
    y_ref[...] = y
    pos_ref[...] = pos

    x = xraw_ref[...]
    nrm = jnp.sqrt(jnp.sum(x * x, axis=1, keepdims=True))
    xn = x / (nrm + np.float32(1e-12))
    rvalid = lax.broadcasted_iota(jnp.int32, x.shape, 0) < NA
    x_ref[...] = jnp.where(rvalid, xn, 0.0)


def _run_prep(ths, xraw, preda2d, cura2d):
    return pl.pallas_call(
        _prep_body,
        out_shape=[
            jax.ShapeDtypeStruct((NA_PAD, DIM), jnp.float32),
            jax.ShapeDtypeStruct((16, 128), jnp.int32),
            jax.ShapeDtypeStruct((16, 128), jnp.int32),
        ],
        in_specs=[
            pl.BlockSpec(memory_space=pltpu.MemorySpace.SMEM),
            pl.BlockSpec(memory_space=pltpu.MemorySpace.VMEM),
            pl.BlockSpec(memory_space=pltpu.MemorySpace.VMEM),
            pl.BlockSpec(memory_space=pltpu.MemorySpace.VMEM),
        ],
    )(ths, xraw, preda2d, cura2d)


def _loss_body(pos_ref, x_ref, q_ref, cc_ref, yrow_ref, ycol_ref,
               out_ref, qmem, acc):
    pid = pl.program_id(0)
    inv_t = np.float32(1.0 / TEMP)

    @pl.when(pid == 0)
    def _init():
        qmem[pl.ds(0, NQ), :] = q_ref[...]
        qmem[pl.ds(NQ, NQ_PAD - NQ), :] = jnp.zeros(
            (NQ_PAD - NQ, DIM), jnp.float32)

        def scatter(i, _):
            p = pos_ref[i]
            qmem[pl.ds(p, 1), :] = x_ref[pl.ds(i, 1), :]
            return 0

        lax.fori_loop(0, NA, scatter, 0)
        acc[0] = 0.0
        acc[1] = 0.0
        acc[2] = 0.0

    r = x_ref[pl.ds(pid * ROW_TILE, ROW_TILE), :] * inv_t
    yr = ycol_ref[pl.ds(pid * ROW_TILE, ROW_TILE), :]
    rowid = pid * ROW_TILE + lax.broadcasted_iota(jnp.int32, (ROW_TILE, 1), 0)
    rvalid = (rowid < NA).astype(jnp.float32)

    a = lax.dot_general(r, x_ref[...], (((1,), (1,)), ((), ())),
                        preferred_element_type=jnp.float32)
    colid1 = lax.broadcasted_iota(jnp.int32, (1, NA_PAD), 1)
    colid = lax.broadcasted_iota(jnp.int32, (ROW_TILE, NA_PAD), 1)
    am = jnp.where(colid1 < NA, a, NEG_BIG)
    m1 = jnp.max(am, axis=1, keepdims=True)
    e1 = jnp.exp(am - m1)
    maskf = (yr == yrow_ref[...]).astype(jnp.float32)
    rowid_b = pid * ROW_TILE + lax.broadcasted_iota(
        jnp.int32, (ROW_TILE, NA_PAD), 0)
    eye = (rowid_b == colid).astype(jnp.float32)
    mask_pos = maskf * (1.0 - eye)
    neg_raw = jnp.sum(e1 * (1.0 - maskf), axis=1, keepdims=True)
    denom = jnp.sum(mask_pos, axis=1, keepdims=True)

    NGRP = QBLK // PIXEL_SIZE + 4
    LOG2E = np.float32(1.4426950408889634)
    m_run = jnp.full((ROW_TILE, 1), NEG_BIG, jnp.float32)
    total = jnp.zeros((ROW_TILE, 1), jnp.float32)
    matched = jnp.zeros((ROW_TILE, 1), jnp.float32)
    for cb in range(N_QBLK):
        qc = lax.dot_general(
            r, qmem[pl.ds(cb * QBLK, QBLK), :], (((1,), (1,)), ((), ())),
            preferred_element_type=jnp.float32)
        if cb == N_QBLK - 1:
            cgl = cb * QBLK + lax.broadcasted_iota(jnp.int32, (1, QBLK), 1)
            qc = jnp.where(cgl < NQ, qc, NEG_BIG)
        m_run = jnp.maximum(m_run, jnp.max(qc, axis=1, keepdims=True))
        e = jnp.exp2(qc * LOG2E)
        gbase = (cb * QBLK) // PIXEL_SIZE
        grow = ((cb * QBLK +
                 lax.broadcasted_iota(jnp.int32, (QBLK, NGRP), 0))
                // PIXEL_SIZE - gbase)
        gsel = (grow == lax.broadcasted_iota(
            jnp.int32, (QBLK, NGRP), 1)).astype(jnp.float32)
        s = lax.dot_general(e, gsel, (((1,), (0,)), ((), ())),
                            preferred_element_type=jnp.float32)
        gg = gbase + lax.broadcasted_iota(jnp.int32, (1, NGRP), 1)
        matched = matched + jnp.sum(jnp.where(yr == gg, s, 0.0),
                                    axis=1, keepdims=True)
        total = total + jnp.sum(s, axis=1, keepdims=True)
    neg_logits = (total - matched) * jnp.exp(-m_run)

    lp = (am - m1) - jnp.log(e1 + neg_logits + neg_raw)
    mlpp = jnp.sum(mask_pos * lp, axis=1, keepdims=True) / jnp.maximum(denom, 1.0)
    validr = (denom > 0).astype(jnp.float32)
    ppc_num = jnp.sum(rvalid * validr * mlpp)
    ppc_cnt = jnp.sum(rvalid * validr)

    a2 = lax.dot_general(r, cc_ref[...], (((1,), (1,)), ((), ())),
                         preferred_element_type=jnp.float32,
                         precision=lax.Precision.HIGHEST)
    c2 = lax.broadcasted_iota(jnp.int32, (1, NUM_CLASSES * KSUB), 1)
    m2 = jnp.max(a2, axis=1, keepdims=True)
    l2 = a2 - m2
    e2 = jnp.exp(l2)
    mask2 = (yr == c2).astype(jnp.float32)
    neg2 = jnp.sum((1.0 - mask2) * e2, axis=1, keepdims=True)
    lp2 = l2 - jnp.log(e2 + neg2)
    d2 = jnp.sum(mask2, axis=1, keepdims=True)
    mlpp2 = jnp.sum(mask2 * lp2, axis=1, keepdims=True) / jnp.maximum(d2, 1.0)
    pcc_num = jnp.sum(rvalid * mlpp2)

    acc[0] += ppc_num
    acc[1] += ppc_cnt
    acc[2] += pcc_num

    @pl.when(pid == N_TILES - 1)
    def _fin():
        scale = np.float32(TEMP / BASE_TEMP)
        loss = (-scale * acc[0] / jnp.maximum(acc[1], 1.0)
                - scale * acc[2] / np.float32(NA))
        out_ref[...] = jnp.full((1, 1), loss, jnp.float32)


def _run_loss(pos1d, x, qpad, ccpad, yrow, ycol):
    grid_spec = pltpu.PrefetchScalarGridSpec(
        num_scalar_prefetch=1,
        grid=(N_TILES,),
        in_specs=[
            pl.BlockSpec((NA_PAD, DIM), lambda i, pos: (0, 0)),
            pl.BlockSpec((NQ, DIM), lambda i, pos: (0, 0)),
            pl.BlockSpec((NUM_CLASSES * KSUB, DIM), lambda i, pos: (0, 0)),
            pl.BlockSpec((1, NA_PAD), lambda i, pos: (0, 0)),
            pl.BlockSpec((NA_PAD, 1), lambda i, pos: (0, 0)),
        ],
        out_specs=pl.BlockSpec((1, 1), lambda i, pos: (0, 0)),
        scratch_shapes=[
            pltpu.VMEM((NQ_PAD, DIM), jnp.float32),
            pltpu.SMEM((4,), jnp.float32),
        ],
    )
    return pl.pallas_call(
        _loss_body,
        grid_spec=grid_spec,
        out_shape=jax.ShapeDtypeStruct((1, 1), jnp.float32),
    )(pos1d, x, qpad, ccpad, yrow, ycol)


def kernel(feats, labels, predict, cur, point_queue, cluster_center,
           anchor_idx):
    del labels
    aidx = anchor_idx.astype(jnp.int32)
    aidx = jnp.concatenate(
        [aidx, jnp.zeros((NA_PAD - NA,), jnp.int32)])
    base = (aidx // 16384) * (DIM * 16384) + (aidx % 16384)
    fidx = (base[:, None] + jnp.arange(DIM, dtype=jnp.int32)[None, :] * 16384)
    fidx = fidx.reshape(NA_PAD * DIM // 128, 128)

    feats_flat = feats.reshape(-1)
    pred_flat = predict.reshape(-1).astype(jnp.int32)
    cur_flat = cur.reshape(-1)

    xraw_flat, preda, cura = _get_sc_gather()(
        fidx, aidx, feats_flat, pred_flat, cur_flat)

    ths = _run_quant(cur_flat.reshape(1024, 128))

    x, y2d, pos2d = _run_prep(
        ths,
        xraw_flat.reshape(NA_PAD, DIM),
        preda.reshape(16, 128),
        cura.reshape(16, 128))

    loss = _run_loss(
        pos2d.reshape(NA_PAD),
        x,
        point_queue.reshape(NQ, DIM),
        cluster_center.reshape(NUM_CLASSES * KSUB, DIM),
        y2d.reshape(1, NA_PAD),
        y2d.reshape(NA_PAD, 1))
    return loss[0, 0]


# speedup vs baseline: 1.0807x; 1.0143x over previous
import functools
import numpy as np
import jax
import jax.numpy as jnp
from jax import lax
from jax.experimental import pallas as pl
from jax.experimental.pallas import tpu as pltpu
from jax.experimental.pallas import tpu_sc as plsc

NUM_CLASSES = 17
KSUB = 6
DIM = 64
PIXEL_SIZE = 150
TEMP = 0.1
BASE_TEMP = 1.0
NA = 2000
NA_PAD = 2048
NPIX = 8 * 128 * 128
NQ = NUM_CLASSES * KSUB * PIXEL_SIZE
NQ_PAD = 15360
NW = 32
APW = NA_PAD // NW
ROW_TILE = 1024
N_TILES = NA_PAD // ROW_TILE
QBLK = 1920
N_QBLK = NQ_PAD // QBLK
NEG_BIG = np.float32(-1e30)

_QS = np.array([0.95, 0.85, 0.75, 0.65, 0.55], dtype=np.float32)
_IDXF = (_QS * np.float32(NPIX - 1)).astype(np.float32)
_LOWF = np.floor(_IDXF).astype(np.float32)
_HIW = (_IDXF - _LOWF).astype(np.float32)
_LOW = (np.float32(1.0) - _HIW).astype(np.float32)
_KS = [int(v) for v in _LOWF]


def _sc_gather_body(idx_hbm, aidx_hbm, feats_hbm, pred_hbm, cur_hbm,
                    xraw_hbm, preda_hbm, cura_hbm,
                    idx_v, rows_v, a_v, pred_v, cur_v, sem, sem2):
    c = lax.axis_index("c")
    s = lax.axis_index("s")
    wid = s * 2 + c
    base = wid * APW
    pltpu.sync_copy(aidx_hbm.at[pl.ds(base, APW)], a_v)
    pltpu.async_copy(pred_hbm.at[a_v], pred_v, sem2).wait()
    pltpu.sync_copy(pred_v, preda_hbm.at[pl.ds(base, APW)])
    pltpu.async_copy(cur_hbm.at[a_v], cur_v, sem2).wait()
    pltpu.sync_copy(cur_v, cura_hbm.at[pl.ds(base, APW)])
    pltpu.sync_copy(idx_hbm.at[pl.ds(wid * 32, 32)], idx_v)
    copies = []
    for j in range(32):
        copies.append(pltpu.async_copy(
            feats_hbm.at[idx_v.at[j]], rows_v.at[pl.ds(j * 128, 128)], sem))
    for cp in copies:
        cp.wait()
    pltpu.sync_copy(rows_v, xraw_hbm.at[pl.ds(wid * 4096, 4096)])


_SC_GATHER_CACHE = []


def _get_sc_gather():
    if not _SC_GATHER_CACHE:
        _SC_GATHER_CACHE.append(functools.partial(
            pl.kernel,
            mesh=plsc.VectorSubcoreMesh(
                core_axis_name="c", subcore_axis_name="s"),
            out_type=[
                jax.ShapeDtypeStruct((NA_PAD * DIM,), jnp.float32),
                jax.ShapeDtypeStruct((NA_PAD,), jnp.int32),
                jax.ShapeDtypeStruct((NA_PAD,), jnp.float32),
            ],
            scratch_types=[
                pltpu.VMEM((32, 128), jnp.int32),
                pltpu.VMEM((4096,), jnp.float32),
                pltpu.VMEM((APW,), jnp.int32),
                pltpu.VMEM((APW,), jnp.int32),
                pltpu.VMEM((APW,), jnp.float32),
                pltpu.SemaphoreType.DMA,
                pltpu.SemaphoreType.DMA,
            ],
        )(_sc_gather_body))
    return _SC_GATHER_CACHE[0]


def _quant_body(cur_ref, ths_ref):
    cur_i = lax.bitcast_convert_type(cur_ref[...], jnp.int32)

    def bs_step(_, carry):
        los, his = carry
        nlos, nhis = [], []
        for t in range(5):
            mid = los[t] + (his[t] - los[t]) // 2
            cnt = jnp.sum((cur_i < mid).astype(jnp.int32))
            ok = cnt <= _KS[t]
            nlos.append(jnp.where(ok, mid, los[t]))
            nhis.append(jnp.where(ok, his[t], mid))
        return tuple(nlos), tuple(nhis)

    zero = jnp.int32(0)
    top = jnp.int32(0x7F800000)
    los, his = lax.fori_loop(
        0, 31, bs_step,
        (tuple(zero for _ in range(5)), tuple(top for _ in range(5))))

    for t in range(5):
        vk = los[t]
        cnt_le = jnp.sum((cur_i <= vk).astype(jnp.int32))
        above = jnp.where(cur_i > vk, cur_i, jnp.int32(0x7F800000))
        vk1 = jnp.where(cnt_le >= _KS[t] + 2, vk, jnp.min(above))
        vkf = lax.bitcast_convert_type(vk, jnp.float32)
        vk1f = lax.bitcast_convert_type(vk1, jnp.float32)
        ths_ref[t] = vkf * np.float32(_LOW[t]) + vk1f * np.float32(_HIW[t])


def _run_quant(cur2d):
    return pl.pallas_call(
        _quant_body,
        out_shape=jax.ShapeDtypeStruct((8,), jnp.float32),
        out_specs=pl.BlockSpec(memory_space=pltpu.MemorySpace.SMEM),
    )(cur2d)


def _prep_body(ths_ref, xraw_ref, preda_ref, cura_ref,
               x_ref, y_ref, pos_ref):
    c = cura_ref[...]
    sub = jnp.zeros(c.shape, jnp.int32)
    for t in range(5):
        sub = sub + (c < ths_ref[t]).astype(jnp.int32)
    ig = (lax.broadcasted_iota(jnp.int32, c.shape, 0) * 128 +
          lax.broadcasted_iota(jnp.int32, c.shape, 1))
    valid = ig < NA
    y = preda_ref[...] * KSUB + sub
    y = jnp.where(valid, y, -1)
    pos = jnp.where(valid, y * PIXEL_SIZE + ig % PIXEL_SIZE, 0)
    y_ref[...] = y
    pos_ref[...] = pos

    x = xraw_ref[...]
    nrm = jnp.sqrt(jnp.sum(x * x, axis=1, keepdims=True))
    xn = x / (nrm + np.float32(1e-12))
    rvalid = lax.broadcasted_iota(jnp.int32, x.shape, 0) < NA
    x_ref[...] = jnp.where(rvalid, xn, 0.0)


def _run_prep(ths, xraw, preda2d, cura2d):
    return pl.pallas_call(
        _prep_body,
        out_shape=[
            jax.ShapeDtypeStruct((NA_PAD, DIM), jnp.float32),
            jax.ShapeDtypeStruct((16, 128), jnp.int32),
            jax.ShapeDtypeStruct((16, 128), jnp.int32),
        ],
        in_specs=[
            pl.BlockSpec(memory_space=pltpu.MemorySpace.SMEM),
            pl.BlockSpec(memory_space=pltpu.MemorySpace.VMEM),
            pl.BlockSpec(memory_space=pltpu.MemorySpace.VMEM),
            pl.BlockSpec(memory_space=pltpu.MemorySpace.VMEM),
        ],
    )(ths, xraw, preda2d, cura2d)


def _loss_body(pos_ref, x_ref, q_ref, cc_ref, yrow_ref, ycol_ref,
               out_ref, qmem, acc):
    pid = pl.program_id(0)
    inv_t = np.float32(1.0 / TEMP)

    @pl.when(pid == 0)
    def _init():
        qmem[pl.ds(0, NQ), :] = q_ref[...]
        qmem[pl.ds(NQ, NQ_PAD - NQ), :] = jnp.zeros(
            (NQ_PAD - NQ, DIM), jnp.float32)

        def scatter(i, _):
            p = pos_ref[i]
            qmem[pl.ds(p, 1), :] = x_ref[pl.ds(i, 1), :]
            return 0

        lax.fori_loop(0, NA, scatter, 0)
        acc[0] = 0.0
        acc[1] = 0.0
        acc[2] = 0.0

    r = x_ref[pl.ds(pid * ROW_TILE, ROW_TILE), :] * inv_t
    yr = ycol_ref[pl.ds(pid * ROW_TILE, ROW_TILE), :]
    rowid = pid * ROW_TILE + lax.broadcasted_iota(jnp.int32, (ROW_TILE, 1), 0)
    rvalid = (rowid < NA).astype(jnp.float32)

    a = lax.dot_general(r, x_ref[...], (((1,), (1,)), ((), ())),
                        preferred_element_type=jnp.float32)
    colid1 = lax.broadcasted_iota(jnp.int32, (1, NA_PAD), 1)
    colid = lax.broadcasted_iota(jnp.int32, (ROW_TILE, NA_PAD), 1)
    am = jnp.where(colid1 < NA, a, NEG_BIG)
    m1 = jnp.max(am, axis=1, keepdims=True)
    e1 = jnp.exp(am - m1)
    maskf = (yr == yrow_ref[...]).astype(jnp.float32)
    rowid_b = pid * ROW_TILE + lax.broadcasted_iota(
        jnp.int32, (ROW_TILE, NA_PAD), 0)
    eye = (rowid_b == colid).astype(jnp.float32)
    mask_pos = maskf * (1.0 - eye)
    neg_raw = jnp.sum(e1 * (1.0 - maskf), axis=1, keepdims=True)
    denom = jnp.sum(mask_pos, axis=1, keepdims=True)

    NGRP = QBLK // PIXEL_SIZE + 4
    LOG2E = np.float32(1.4426950408889634)
    m_run = jnp.full((ROW_TILE, 1), NEG_BIG, jnp.float32)
    total = jnp.zeros((ROW_TILE, 1), jnp.float32)
    matched = jnp.zeros((ROW_TILE, 1), jnp.float32)
    for cb in range(N_QBLK):
        qc = lax.dot_general(
            r, qmem[pl.ds(cb * QBLK, QBLK), :], (((1,), (1,)), ((), ())),
            preferred_element_type=jnp.float32)
        if cb == N_QBLK - 1:
            cgl = cb * QBLK + lax.broadcasted_iota(jnp.int32, (1, QBLK), 1)
            qc = jnp.where(cgl < NQ, qc, NEG_BIG)
        m_run = jnp.maximum(m_run, jnp.max(qc, axis=1, keepdims=True))
        e = jnp.exp2(qc * LOG2E)
        gbase = (cb * QBLK) // PIXEL_SIZE
        grow = ((cb * QBLK +
                 lax.broadcasted_iota(jnp.int32, (QBLK, NGRP), 0))
                // PIXEL_SIZE - gbase)
        gsel = (grow == lax.broadcasted_iota(
            jnp.int32, (QBLK, NGRP), 1)).astype(jnp.float32)
        s = lax.dot_general(e, gsel, (((1,), (0,)), ((), ())),
                            preferred_element_type=jnp.float32)
        gg = gbase + lax.broadcasted_iota(jnp.int32, (1, NGRP), 1)
        matched = matched + jnp.sum(jnp.where(yr == gg, s, 0.0),
                                    axis=1, keepdims=True)
        total = total + jnp.sum(s, axis=1, keepdims=True)
    neg_logits = (total - matched) * jnp.exp(-m_run)

    lp = (am - m1) - jnp.log(e1 + neg_logits + neg_raw)
    mlpp = jnp.sum(mask_pos * lp, axis=1, keepdims=True) / jnp.maximum(denom, 1.0)
    validr = (denom > 0).astype(jnp.float32)
    ppc_num = jnp.sum(rvalid * validr * mlpp)
    ppc_cnt = jnp.sum(rvalid * validr)

    a2 = lax.dot_general(r, cc_ref[...], (((1,), (1,)), ((), ())),
                         preferred_element_type=jnp.float32,
                         precision=lax.Precision.HIGHEST)
    c2 = lax.broadcasted_iota(jnp.int32, (1, NUM_CLASSES * KSUB), 1)
    m2 = jnp.max(a2, axis=1, keepdims=True)
    l2 = a2 - m2
    e2 = jnp.exp(l2)
    mask2 = (yr == c2).astype(jnp.float32)
    neg2 = jnp.sum((1.0 - mask2) * e2, axis=1, keepdims=True)
    lp2 = l2 - jnp.log(e2 + neg2)
    d2 = jnp.sum(mask2, axis=1, keepdims=True)
    mlpp2 = jnp.sum(mask2 * lp2, axis=1, keepdims=True) / jnp.maximum(d2, 1.0)
    pcc_num = jnp.sum(rvalid * mlpp2)

    acc[0] += ppc_num
    acc[1] += ppc_cnt
    acc[2] += pcc_num

    @pl.when(pid == N_TILES - 1)
    def _fin():
        scale = np.float32(TEMP / BASE_TEMP)
        loss = (-scale * acc[0] / jnp.maximum(acc[1], 1.0)
                - scale * acc[2] / np.float32(NA))
        out_ref[...] = jnp.full((1, 1), loss, jnp.float32)


def _run_loss(pos1d, x, qpad, ccpad, yrow, ycol):
    grid_spec = pltpu.PrefetchScalarGridSpec(
        num_scalar_prefetch=1,
        grid=(N_TILES,),
        in_specs=[
            pl.BlockSpec((NA_PAD, DIM), lambda i, pos: (0, 0)),
            pl.BlockSpec((NQ, DIM), lambda i, pos: (0, 0)),
            pl.BlockSpec((NUM_CLASSES * KSUB, DIM), lambda i, pos: (0, 0)),
            pl.BlockSpec((1, NA_PAD), lambda i, pos: (0, 0)),
            pl.BlockSpec((NA_PAD, 1), lambda i, pos: (0, 0)),
        ],
        out_specs=pl.BlockSpec((1, 1), lambda i, pos: (0, 0)),
        scratch_shapes=[
            pltpu.VMEM((NQ_PAD, DIM), jnp.float32),
            pltpu.SMEM((4,), jnp.float32),
        ],
    )
    return pl.pallas_call(
        _loss_body,
        grid_spec=grid_spec,
        out_shape=jax.ShapeDtypeStruct((1, 1), jnp.float32),
    )(pos1d, x, qpad, ccpad, yrow, ycol)


def kernel(feats, labels, predict, cur, point_queue, cluster_center,
           anchor_idx):
    del labels
    aidx = anchor_idx.astype(jnp.int32)
    aidx = jnp.concatenate(
        [aidx, jnp.zeros((NA_PAD - NA,), jnp.int32)])
    base = (aidx // 16384) * (DIM * 16384) + (aidx % 16384)
    fidx = (base[:, None] + jnp.arange(DIM, dtype=jnp.int32)[None, :] * 16384)
    fidx = fidx.reshape(NA_PAD * DIM // 128, 128)

    feats_flat = feats.reshape(-1)
    pred_flat = predict.reshape(-1).astype(jnp.int32)
    cur_flat = cur.reshape(-1)

    xraw_flat, preda, cura = _get_sc_gather()(
        fidx, aidx, feats_flat, pred_flat, cur_flat)

    ths = _run_quant(cur_flat.reshape(1024, 128))

    x, y2d, pos2d = _run_prep(
        ths,
        xraw_flat.reshape(NA_PAD, DIM),
        preda.reshape(16, 128),
        cura.reshape(16, 128))

    loss = _run_loss(
        pos2d.reshape(NA_PAD),
        x,
        point_queue.reshape(NQ, DIM),
        cluster_center.reshape(NUM_CLASSES * KSUB, DIM),
        y2d.reshape(1, NA_PAD),
        y2d.reshape(NA_PAD, 1))
    return loss[0, 0]


# speedup vs baseline: 1.0923x; 1.0107x over previous
import functools
import numpy as np
import jax
import jax.numpy as jnp
from jax import lax
from jax.experimental import pallas as pl
from jax.experimental.pallas import tpu as pltpu
from jax.experimental.pallas import tpu_sc as plsc

NUM_CLASSES = 17
KSUB = 6
DIM = 64
PIXEL_SIZE = 150
TEMP = 0.1
BASE_TEMP = 1.0
NA = 2000
NA_PAD = 2048
NPIX = 8 * 128 * 128
NQ = NUM_CLASSES * KSUB * PIXEL_SIZE
NQ_PAD = 15360
NW = 32
APW = NA_PAD // NW
ROW_TILE = 1024
N_TILES = NA_PAD // ROW_TILE
QBLK = 1920
N_QBLK = NQ_PAD // QBLK
NEG_BIG = np.float32(-1e30)

_QS = np.array([0.95, 0.85, 0.75, 0.65, 0.55], dtype=np.float32)
_IDXF = (_QS * np.float32(NPIX - 1)).astype(np.float32)
_LOWF = np.floor(_IDXF).astype(np.float32)
_HIW = (_IDXF - _LOWF).astype(np.float32)
_LOW = (np.float32(1.0) - _HIW).astype(np.float32)
_KS = [int(v) for v in _LOWF]


def _sc_gather_body(idx_hbm, aidx_hbm, feats_hbm, pred_hbm, cur_hbm,
                    xraw_hbm, preda_hbm, cura_hbm,
                    idx_v, rows_v, a_v, pred_v, cur_v, sem, sem2):
    c = lax.axis_index("c")
    s = lax.axis_index("s")
    wid = s * 2 + c
    base = wid * APW
    pltpu.sync_copy(aidx_hbm.at[pl.ds(base, APW)], a_v)
    pltpu.async_copy(pred_hbm.at[a_v], pred_v, sem2).wait()
    pltpu.sync_copy(pred_v, preda_hbm.at[pl.ds(base, APW)])
    pltpu.async_copy(cur_hbm.at[a_v], cur_v, sem2).wait()
    pltpu.sync_copy(cur_v, cura_hbm.at[pl.ds(base, APW)])
    pltpu.sync_copy(idx_hbm.at[pl.ds(wid * 32, 32)], idx_v)
    copies = []
    for j in range(32):
        copies.append(pltpu.async_copy(
            feats_hbm.at[idx_v.at[j]], rows_v.at[pl.ds(j * 128, 128)], sem))
    for cp in copies:
        cp.wait()
    pltpu.sync_copy(rows_v, xraw_hbm.at[pl.ds(wid * 4096, 4096)])


_SC_GATHER_CACHE = []


def _get_sc_gather():
    if not _SC_GATHER_CACHE:
        _SC_GATHER_CACHE.append(functools.partial(
            pl.kernel,
            mesh=plsc.VectorSubcoreMesh(
                core_axis_name="c", subcore_axis_name="s"),
            out_type=[
                jax.ShapeDtypeStruct((NA_PAD * DIM,), jnp.float32),
                jax.ShapeDtypeStruct((NA_PAD,), jnp.int32),
                jax.ShapeDtypeStruct((NA_PAD,), jnp.float32),
            ],
            scratch_types=[
                pltpu.VMEM((32, 128), jnp.int32),
                pltpu.VMEM((4096,), jnp.float32),
                pltpu.VMEM((APW,), jnp.int32),
                pltpu.VMEM((APW,), jnp.int32),
                pltpu.VMEM((APW,), jnp.float32),
                pltpu.SemaphoreType.DMA,
                pltpu.SemaphoreType.DMA,
            ],
        )(_sc_gather_body))
    return _SC_GATHER_CACHE[0]


def _quant_body(cur_ref, ths_ref):
    cur_i = lax.bitcast_convert_type(cur_ref[...], jnp.int32)
    cur3 = cur_i[None, :, :]
    ti = lax.broadcasted_iota(jnp.int32, (5, 1, 1), 0)
    ksv = jnp.zeros((5, 1, 1), jnp.int32)
    lowv = jnp.zeros((5, 1, 1), jnp.float32)
    hiwv = jnp.zeros((5, 1, 1), jnp.float32)
    for t in range(5):
        sel = ti == t
        ksv = jnp.where(sel, np.int32(_KS[t]), ksv)
        lowv = jnp.where(sel, np.float32(_LOW[t]), lowv)
        hiwv = jnp.where(sel, np.float32(_HIW[t]), hiwv)

    def bs_step(_, carry):
        los, his = carry
        mid = los + (his - los) // 2
        cnt = jnp.sum((cur3 < mid).astype(jnp.int32),
                      axis=(1, 2), keepdims=True)
        ok = cnt <= ksv
        return jnp.where(ok, mid, los), jnp.where(ok, his, mid)

    los, his = lax.fori_loop(
        0, 31, bs_step,
        (jnp.zeros((5, 1, 1), jnp.int32),
         jnp.full((5, 1, 1), 0x7F800000, jnp.int32)))

    cnt_le = jnp.sum((cur3 <= los).astype(jnp.int32),
                     axis=(1, 2), keepdims=True)
    above = jnp.where(cur3 > los, cur3, jnp.int32(0x7F800000))
    mina = jnp.min(above, axis=(1, 2), keepdims=True)
    vk1 = jnp.where(cnt_le >= ksv + 2, los, mina)
    vkf = lax.bitcast_convert_type(los, jnp.float32)
    vk1f = lax.bitcast_convert_type(vk1, jnp.float32)
    ths = vkf * lowv + vk1f * hiwv
    ths8 = jnp.concatenate([ths, jnp.zeros((3, 1, 1), jnp.float32)])
    ths_ref[...] = jnp.broadcast_to(ths8.reshape(8, 1), (8, 128))


def _run_quant(cur2d):
    return pl.pallas_call(
        _quant_body,
        out_shape=jax.ShapeDtypeStruct((8, 128), jnp.float32),
    )(cur2d)


def _prep_body(ths_ref, xraw_ref, preda_ref, cura_ref,
               x_ref, y_ref, pos_ref):
    c = cura_ref[...]
    sub = jnp.zeros(c.shape, jnp.int32)
    for t in range(5):
        sub = sub + (c < ths_ref[t:t + 1, :]).astype(jnp.int32)
    ig = (lax.broadcasted_iota(jnp.int32, c.shape, 0) * 128 +
          lax.broadcasted_iota(jnp.int32, c.shape, 1))
    valid = ig < NA
    y = preda_ref[...] * KSUB + sub
    y = jnp.where(valid, y, -1)
    pos = jnp.where(valid, y * PIXEL_SIZE + ig % PIXEL_SIZE, 0)
    y_ref[...] = y
    pos_ref[...] = pos

    x = xraw_ref[...]
    nrm = jnp.sqrt(jnp.sum(x * x, axis=1, keepdims=True))
    xn = x / (nrm + np.float32(1e-12))
    rvalid = lax.broadcasted_iota(jnp.int32, x.shape, 0) < NA
    x_ref[...] = jnp.where(rvalid, xn, 0.0)


def _run_prep(ths, xraw, preda2d, cura2d):
    return pl.pallas_call(
        _prep_body,
        out_shape=[
            jax.ShapeDtypeStruct((NA_PAD, DIM), jnp.float32),
            jax.ShapeDtypeStruct((16, 128), jnp.int32),
            jax.ShapeDtypeStruct((16, 128), jnp.int32),
        ],
    )(ths, xraw, preda2d, cura2d)


def _loss_body(pos_ref, x_ref, q_ref, cc_ref, yrow_ref, ycol_ref,
               out_ref, qmem, acc):
    pid = pl.program_id(0)
    inv_t = np.float32(1.0 / TEMP)

    @pl.when(pid == 0)
    def _init():
        qmem[pl.ds(0, NQ), :] = q_ref[...]
        qmem[pl.ds(NQ, NQ_PAD - NQ), :] = jnp.zeros(
            (NQ_PAD - NQ, DIM), jnp.float32)

        def scatter(i, _):
            p = pos_ref[i]
            qmem[pl.ds(p, 1), :] = x_ref[pl.ds(i, 1), :]
            return 0

        lax.fori_loop(0, NA, scatter, 0)
        acc[0] = 0.0
        acc[1] = 0.0
        acc[2] = 0.0

    r = x_ref[pl.ds(pid * ROW_TILE, ROW_TILE), :] * inv_t
    yr = ycol_ref[pl.ds(pid * ROW_TILE, ROW_TILE), :]
    rowid = pid * ROW_TILE + lax.broadcasted_iota(jnp.int32, (ROW_TILE, 1), 0)
    rvalid = (rowid < NA).astype(jnp.float32)

    a = lax.dot_general(r, x_ref[...], (((1,), (1,)), ((), ())),
                        preferred_element_type=jnp.float32)
    colid1 = lax.broadcasted_iota(jnp.int32, (1, NA_PAD), 1)
    colid = lax.broadcasted_iota(jnp.int32, (ROW_TILE, NA_PAD), 1)
    am = jnp.where(colid1 < NA, a, NEG_BIG)
    m1 = jnp.max(am, axis=1, keepdims=True)
    e1 = jnp.exp(am - m1)
    maskf = (yr == yrow_ref[...]).astype(jnp.float32)
    rowid_b = pid * ROW_TILE + lax.broadcasted_iota(
        jnp.int32, (ROW_TILE, NA_PAD), 0)
    eye = (rowid_b == colid).astype(jnp.float32)
    mask_pos = maskf * (1.0 - eye)
    neg_raw = jnp.sum(e1 * (1.0 - maskf), axis=1, keepdims=True)
    denom = jnp.sum(mask_pos, axis=1, keepdims=True)

    NGRP = QBLK // PIXEL_SIZE + 4
    LOG2E = np.float32(1.4426950408889634)
    m_run = jnp.full((ROW_TILE, 1), NEG_BIG, jnp.float32)
    total = jnp.zeros((ROW_TILE, 1), jnp.float32)
    matched = jnp.zeros((ROW_TILE, 1), jnp.float32)
    for cb in range(N_QBLK):
        qc = lax.dot_general(
            r, qmem[pl.ds(cb * QBLK, QBLK), :], (((1,), (1,)), ((), ())),
            preferred_element_type=jnp.float32)
        if cb == N_QBLK - 1:
            cgl = cb * QBLK + lax.broadcasted_iota(jnp.int32, (1, QBLK), 1)
            qc = jnp.where(cgl < NQ, qc, NEG_BIG)
        m_run = jnp.maximum(m_run, jnp.max(qc, axis=1, keepdims=True))
        e = jnp.exp2(qc * LOG2E)
        gbase = (cb * QBLK) // PIXEL_SIZE
        grow = ((cb * QBLK +
                 lax.broadcasted_iota(jnp.int32, (QBLK, NGRP), 0))
                // PIXEL_SIZE - gbase)
        gsel = (grow == lax.broadcasted_iota(
            jnp.int32, (QBLK, NGRP), 1)).astype(jnp.float32)
        s = lax.dot_general(e, gsel, (((1,), (0,)), ((), ())),
                            preferred_element_type=jnp.float32)
        gg = gbase + lax.broadcasted_iota(jnp.int32, (1, NGRP), 1)
        matched = matched + jnp.sum(jnp.where(yr == gg, s, 0.0),
                                    axis=1, keepdims=True)
        total = total + jnp.sum(s, axis=1, keepdims=True)
    neg_logits = (total - matched) * jnp.exp(-m_run)

    lp = (am - m1) - jnp.log(e1 + neg_logits + neg_raw)
    mlpp = jnp.sum(mask_pos * lp, axis=1, keepdims=True) / jnp.maximum(denom, 1.0)
    validr = (denom > 0).astype(jnp.float32)
    ppc_num = jnp.sum(rvalid * validr * mlpp)
    ppc_cnt = jnp.sum(rvalid * validr)

    a2 = lax.dot_general(r, cc_ref[...], (((1,), (1,)), ((), ())),
                         preferred_element_type=jnp.float32,
                         precision=lax.Precision.HIGHEST)
    c2 = lax.broadcasted_iota(jnp.int32, (1, NUM_CLASSES * KSUB), 1)
    m2 = jnp.max(a2, axis=1, keepdims=True)
    l2 = a2 - m2
    e2 = jnp.exp(l2)
    mask2 = (yr == c2).astype(jnp.float32)
    neg2 = jnp.sum((1.0 - mask2) * e2, axis=1, keepdims=True)
    lp2 = l2 - jnp.log(e2 + neg2)
    d2 = jnp.sum(mask2, axis=1, keepdims=True)
    mlpp2 = jnp.sum(mask2 * lp2, axis=1, keepdims=True) / jnp.maximum(d2, 1.0)
    pcc_num = jnp.sum(rvalid * mlpp2)

    acc[0] += ppc_num
    acc[1] += ppc_cnt
    acc[2] += pcc_num

    @pl.when(pid == N_TILES - 1)
    def _fin():
        scale = np.float32(TEMP / BASE_TEMP)
        loss = (-scale * acc[0] / jnp.maximum(acc[1], 1.0)
                - scale * acc[2] / np.float32(NA))
        out_ref[...] = jnp.full((1, 1), loss, jnp.float32)


def _run_loss(pos1d, x, qpad, ccpad, yrow, ycol):
    grid_spec = pltpu.PrefetchScalarGridSpec(
        num_scalar_prefetch=1,
        grid=(N_TILES,),
        in_specs=[
            pl.BlockSpec((NA_PAD, DIM), lambda i, pos: (0, 0)),
            pl.BlockSpec((NQ, DIM), lambda i, pos: (0, 0)),
            pl.BlockSpec((NUM_CLASSES * KSUB, DIM), lambda i, pos: (0, 0)),
            pl.BlockSpec((1, NA_PAD), lambda i, pos: (0, 0)),
            pl.BlockSpec((NA_PAD, 1), lambda i, pos: (0, 0)),
        ],
        out_specs=pl.BlockSpec((1, 1), lambda i, pos: (0, 0)),
        scratch_shapes=[
            pltpu.VMEM((NQ_PAD, DIM), jnp.float32),
            pltpu.SMEM((4,), jnp.float32),
        ],
    )
    return pl.pallas_call(
        _loss_body,
        grid_spec=grid_spec,
        out_shape=jax.ShapeDtypeStruct((1, 1), jnp.float32),
    )(pos1d, x, qpad, ccpad, yrow, ycol)


def kernel(feats, labels, predict, cur, point_queue, cluster_center,
           anchor_idx):
    del labels
    aidx = anchor_idx.astype(jnp.int32)
    aidx = jnp.concatenate(
        [aidx, jnp.zeros((NA_PAD - NA,), jnp.int32)])
    base = (aidx // 16384) * (DIM * 16384) + (aidx % 16384)
    fidx = (base[:, None] + jnp.arange(DIM, dtype=jnp.int32)[None, :] * 16384)
    fidx = fidx.reshape(NA_PAD * DIM // 128, 128)

    feats_flat = feats.reshape(-1)
    pred_flat = predict.reshape(-1).astype(jnp.int32)
    cur_flat = cur.reshape(-1)

    xraw_flat, preda, cura = _get_sc_gather()(
        fidx, aidx, feats_flat, pred_flat, cur_flat)

    ths = _run_quant(cur_flat.reshape(1024, 128))

    x, y2d, pos2d = _run_prep(
        ths,
        xraw_flat.reshape(NA_PAD, DIM),
        preda.reshape(16, 128),
        cura.reshape(16, 128))

    loss = _run_loss(
        pos2d.reshape(NA_PAD),
        x,
        point_queue.reshape(NQ, DIM),
        cluster_center.reshape(NUM_CLASSES * KSUB, DIM),
        y2d.reshape(1, NA_PAD),
        y2d.reshape(NA_PAD, 1))
    return loss[0, 0]
